# baseline (speedup 1.0000x reference)
import functools

import jax
import jax.numpy as jnp
from jax import lax
from jax.experimental import pallas as pl
from jax.experimental.pallas import tpu as pltpu
from jax.experimental.pallas import tpu_sc as plsc

_N = 10000
_E = 320000
_H = 64
_G = 64

_NC = 2
_NS = 16
_NW = _NC * _NS
_C = 200
_NCHUNK = _E // _C
_K = _NCHUNK // _NW
_N_PAD = 10112
_ZR = _N_PAD // _NS

_sc_mesh = plsc.VectorSubcoreMesh(core_axis_name="c", subcore_axis_name="s")


@functools.partial(
    pl.kernel,
    mesh=_sc_mesh,
    compiler_params=pltpu.CompilerParams(use_tc_tiling_on_sc=False),
    out_type=jax.ShapeDtypeStruct((_NC, _N_PAD, _H), jnp.float32),
    scratch_types=[
        pltpu.VMEM_SHARED((_N_PAD, _H), jnp.float32),
        pltpu.VMEM_SHARED((_N_PAD, _H), jnp.float32),
        pltpu.VMEM((_K, _C), jnp.int32),
        pltpu.VMEM((_K, _C), jnp.int32),
        pltpu.VMEM((2, _C, _H), jnp.float32),
        pltpu.SemaphoreType.DMA,
        pltpu.SemaphoreType.DMA,
    ],
)
def _sc_agg(h_hbm, src_hbm, dst_hbm, zeros_hbm, out_hbm,
            acc, h_s, src_v, dst_v, rows, sem, sem_st):
    cid = lax.axis_index("c")
    sid = lax.axis_index("s")
    wid = sid * _NC + cid

    pltpu.sync_copy(src_hbm.at[pl.ds(wid * _K, _K)], src_v)
    pltpu.sync_copy(dst_hbm.at[pl.ds(wid * _K, _K)], dst_v)
    _LAST = _N - (_NS - 1) * _ZR

    @pl.when(sid < _NS - 1)
    def _():
        pltpu.async_copy(h_hbm.at[pl.ds(sid * _ZR, _ZR)],
                         h_s.at[pl.ds(sid * _ZR, _ZR)], sem_st)

    @pl.when(sid == _NS - 1)
    def _():
        pltpu.async_copy(h_hbm.at[pl.ds((_NS - 1) * _ZR, _LAST)],
                         h_s.at[pl.ds((_NS - 1) * _ZR, _LAST)], sem_st)

    pltpu.async_copy(zeros_hbm.at[pl.ds(sid * _ZR, _ZR)],
                     acc.at[pl.ds(sid * _ZR, _ZR)], sem_st)

    @pl.when(sid < _NS - 1)
    def _():
        pltpu.make_async_copy(h_hbm.at[pl.ds(0, _ZR)],
                              h_s.at[pl.ds(0, _ZR)], sem_st).wait()

    @pl.when(sid == _NS - 1)
    def _():
        pltpu.make_async_copy(h_hbm.at[pl.ds(0, _LAST)],
                              h_s.at[pl.ds(0, _LAST)], sem_st).wait()

    pltpu.make_async_copy(zeros_hbm.at[pl.ds(0, _ZR)],
                          acc.at[pl.ds(0, _ZR)], sem_st).wait()
    plsc.subcore_barrier()

    pltpu.async_copy(h_s.at[src_v.at[0]], rows.at[0], sem)

    def body(j, carry):
        p = lax.rem(j, 2)

        @pl.when(j + 1 < _K)
        def _():
            pltpu.async_copy(h_s.at[src_v.at[j + 1]], rows.at[1 - p], sem)

        pltpu.make_async_copy(h_hbm.at[pl.ds(0, _C)], rows.at[p], sem).wait()
        pltpu.sync_copy(rows.at[p], acc.at[dst_v.at[j]], add=True)
        return carry

    lax.fori_loop(0, _K, body, 0)
    plsc.subcore_barrier()
    pltpu.sync_copy(acc.at[pl.ds(sid * _ZR, _ZR)],
                    out_hbm.at[cid, pl.ds(sid * _ZR, _ZR)])


def _tc_in(x_ref, w_ref, b_ref, o_ref):
    o_ref[...] = jnp.maximum(
        jnp.dot(x_ref[...], w_ref[...], preferred_element_type=jnp.float32)
        + b_ref[...], 0.0)


def _tc_mlp(h_ref, agg_ref, w1_ref, b1_ref, w2_ref, b2_ref, o_ref):
    z = h_ref[...] + agg_ref[0, :_N] + agg_ref[1, :_N]
    z = jnp.maximum(
        jnp.dot(z, w1_ref[...], preferred_element_type=jnp.float32)
        + b1_ref[...], 0.0)
    z = jnp.dot(z, w2_ref[...], preferred_element_type=jnp.float32) + b2_ref[...]
    o_ref[...] = jnp.maximum(z, 0.0)


def _tc_tail(h_ref, agg_ref, batch_ref, w1_ref, b1_ref, w2_ref, b2_ref,
             wo_ref, bo_ref, hg_ref, pred_ref):
    z = h_ref[...] + agg_ref[0, :_N] + agg_ref[1, :_N]
    z = jnp.maximum(
        jnp.dot(z, w1_ref[...], preferred_element_type=jnp.float32)
        + b1_ref[...], 0.0)
    z = jnp.dot(z, w2_ref[...], preferred_element_type=jnp.float32) + b2_ref[...]
    h2 = jnp.maximum(z, 0.0)
    onehot = (batch_ref[...] ==
              lax.broadcasted_iota(jnp.int32, (_N, _G), 1)).astype(jnp.float32)
    sums = lax.dot_general(onehot, h2, (((0,), (0,)), ((), ())),
                           preferred_element_type=jnp.float32)
    counts = jnp.sum(onehot, axis=0)
    hg = sums / jnp.maximum(counts, 1.0)[:, None]
    hg_ref[...] = hg
    pred_ref[...] = (
        jnp.dot(hg, wo_ref[...], preferred_element_type=jnp.float32)
        + bo_ref[...])


def kernel(x, edge_index, batch, W_in, b_in, W1_0, b1_0, W2_0, b2_0,
           W1_1, b1_1, W2_1, b2_1, W_exp, b_exp, W_aux, b_aux):
    f32 = jnp.float32
    src_p = edge_index[0].reshape(_NCHUNK, _C)
    dst_p = edge_index[1].reshape(_NCHUNK, _C)
    zeros = jnp.zeros((_N_PAD, _H), f32)

    h0 = pl.pallas_call(
        _tc_in,
        out_shape=jax.ShapeDtypeStruct((_N, _H), f32),
    )(x, W_in, b_in.reshape(1, _H))

    agg0 = _sc_agg(h0, src_p, dst_p, zeros)

    h1 = pl.pallas_call(
        _tc_mlp,
        out_shape=jax.ShapeDtypeStruct((_N, _H), f32),
    )(h0, agg0, W1_0, b1_0.reshape(1, _H), W2_0, b2_0.reshape(1, _H))

    agg1 = _sc_agg(h1, src_p, dst_p, zeros)

    W_out = jnp.concatenate([W_exp, W_aux], axis=1)
    b_out = jnp.concatenate([b_exp, b_aux]).reshape(1, 5)
    hg, preds = pl.pallas_call(
        _tc_tail,
        out_shape=(jax.ShapeDtypeStruct((_G, _H), f32),
                   jax.ShapeDtypeStruct((_G, 5), f32)),
    )(h1, agg1, batch.reshape(_N, 1), W1_1, b1_1.reshape(1, _H),
      W2_1, b2_1.reshape(1, _H), W_out, b_out)

    return (hg, preds[:, 0:1], preds[:, 1:5])
